# baseline (speedup 1.0000x reference)
import jax
import jax.numpy as jnp
from jax.experimental import pallas as pl

BT = 1024


def _body(x_ref, wf_ref, bf_ref, wr_ref, br_ref, w2_ref, lb_ref, out_ref):
    bt = x_ref.shape[0]
    n_leaf, n_cls = lb_ref.shape
    d_f = wf_ref.shape[1]

    x = x_ref[...]
    feat = jnp.maximum(
        jnp.dot(x, wf_ref[...], preferred_element_type=jnp.float32) + bf_ref[...],
        0.0,
    )
    logits = jnp.dot(x, wr_ref[...], preferred_element_type=jnp.float32) + br_ref[...]
    lmax = jnp.max(logits, axis=1, keepdims=True)
    eidx = jax.lax.broadcasted_iota(jnp.int32, (bt, n_leaf), 1)
    choices = jnp.min(jnp.where(logits == lmax, eidx, n_leaf), axis=1, keepdims=True)
    onehot = (eidx == choices).astype(jnp.float32)

    acc = jnp.dot(onehot, lb_ref[...], preferred_element_type=jnp.float32)
    for e in range(n_leaf):
        pe = jnp.dot(feat, w2_ref[pl.ds(e * d_f, d_f), :],
                     preferred_element_type=jnp.float32)
        acc = acc + jnp.where(choices == e, pe, 0.0)
    out_ref[...] = acc


def kernel(inputs, Wf, bf, Wr, br, leaf_W, leaf_b):
    n_tok, d_in = inputs.shape
    d_f = Wf.shape[1]
    n_leaf, _, n_cls = leaf_W.shape
    w2 = leaf_W.reshape(n_leaf * d_f, n_cls)
    grid = (n_tok // BT,)
    return pl.pallas_call(
        _body,
        grid=grid,
        in_specs=[
            pl.BlockSpec((BT, d_in), lambda i: (i, 0)),
            pl.BlockSpec((d_in, d_f), lambda i: (0, 0)),
            pl.BlockSpec((1, d_f), lambda i: (0, 0)),
            pl.BlockSpec((d_in, n_leaf), lambda i: (0, 0)),
            pl.BlockSpec((1, n_leaf), lambda i: (0, 0)),
            pl.BlockSpec((n_leaf * d_f, n_cls), lambda i: (0, 0)),
            pl.BlockSpec((n_leaf, n_cls), lambda i: (0, 0)),
        ],
        out_specs=pl.BlockSpec((BT, n_cls), lambda i: (i, 0)),
        out_shape=jax.ShapeDtypeStruct((n_tok, n_cls), jnp.float32),
    )(inputs, Wf, bf.reshape(1, d_f), Wr, br.reshape(1, n_leaf), w2, leaf_b)


# speedup vs baseline: 1.0344x; 1.0344x over previous
import jax
import jax.numpy as jnp
from jax.experimental import pallas as pl

BT = 512


def _body(x_ref, wf_ref, bf_ref, wr_ref, br_ref, w2_ref, lb_ref, out_ref):
    bt = x_ref.shape[0]
    n_leaf, n_cls = lb_ref.shape
    d_f = wf_ref.shape[1]

    x = x_ref[...]
    feat = jnp.maximum(
        jnp.dot(x, wf_ref[...], preferred_element_type=jnp.float32) + bf_ref[...],
        0.0,
    )
    logits = jnp.dot(x, wr_ref[...], preferred_element_type=jnp.float32) + br_ref[...]
    lmax = jnp.max(logits, axis=1, keepdims=True)
    eidx = jax.lax.broadcasted_iota(jnp.int32, (bt, n_leaf), 1)
    choices = jnp.min(jnp.where(logits == lmax, eidx, n_leaf), axis=1, keepdims=True)
    onehot = (eidx == choices).astype(jnp.float32)

    acc = jnp.dot(onehot, lb_ref[...], preferred_element_type=jnp.float32)
    for e in range(n_leaf):
        pe = jnp.dot(feat, w2_ref[pl.ds(e * d_f, d_f), :],
                     preferred_element_type=jnp.float32)
        acc = acc + jnp.where(choices == e, pe, 0.0)
    out_ref[...] = acc


def kernel(inputs, Wf, bf, Wr, br, leaf_W, leaf_b):
    n_tok, d_in = inputs.shape
    d_f = Wf.shape[1]
    n_leaf, _, n_cls = leaf_W.shape
    w2 = leaf_W.reshape(n_leaf * d_f, n_cls)
    grid = (n_tok // BT,)
    return pl.pallas_call(
        _body,
        grid=grid,
        in_specs=[
            pl.BlockSpec((BT, d_in), lambda i: (i, 0)),
            pl.BlockSpec((d_in, d_f), lambda i: (0, 0)),
            pl.BlockSpec((1, d_f), lambda i: (0, 0)),
            pl.BlockSpec((d_in, n_leaf), lambda i: (0, 0)),
            pl.BlockSpec((1, n_leaf), lambda i: (0, 0)),
            pl.BlockSpec((n_leaf * d_f, n_cls), lambda i: (0, 0)),
            pl.BlockSpec((n_leaf, n_cls), lambda i: (0, 0)),
        ],
        out_specs=pl.BlockSpec((BT, n_cls), lambda i: (i, 0)),
        out_shape=jax.ShapeDtypeStruct((n_tok, n_cls), jnp.float32),
    )(inputs, Wf, bf.reshape(1, d_f), Wr, br.reshape(1, n_leaf), w2, leaf_b)
